# baseline (speedup 1.0000x reference)
import functools

import jax
import jax.numpy as jnp
from jax import lax
from jax.experimental import pallas as pl
from jax.experimental.pallas import tpu as pltpu

N_NODES = 10000
N_EDGES = 160000
D_EMBED = 256
ATTR_DIM = 16
HIDDEN = 64
IN_DIM = ATTR_DIM + 3

ROW_BLOCK = 1000
N_BLOCKS = N_NODES // ROW_BLOCK


def _struct_feats(edge_index, num_nodes):
    src = edge_index[0].astype(jnp.int32)
    dst = edge_index[1].astype(jnp.int32)
    num_edges = src.shape[0]
    deg = jax.ops.segment_sum(jnp.ones((num_edges,), jnp.float32), src,
                              num_segments=num_nodes)
    BIG = jnp.iinfo(jnp.int32).max
    pos = jnp.arange(num_edges, dtype=jnp.int32)
    dist0 = jnp.full((num_nodes,), -1, jnp.int32).at[0].set(0)
    rank0 = jnp.full((num_nodes,), BIG, jnp.int32).at[0].set(0)
    parent0 = jnp.full((num_nodes,), -1, jnp.int32).at[0].set(0)

    def bfs_cond(c):
        return c[4] > 0

    def bfs_body(c):
        dist, rank, parent, level, _, next_rank = c
        cand = (dist[src] == level) & (dist[dst] < 0)
        key = jnp.where(cand, rank[src] * num_edges + pos, BIG)
        mink = jax.ops.segment_min(key, dst, num_segments=num_nodes)
        new = mink < BIG
        win = cand & (key == mink[dst])
        psrc = jax.ops.segment_max(jnp.where(win, src, -1), dst,
                                   num_segments=num_nodes)
        order = jnp.argsort(jnp.where(new, mink, BIG))
        slot = jnp.argsort(order).astype(jnp.int32)
        rank_new = jnp.where(new, next_rank + slot, rank)
        dist_new = jnp.where(new, level + 1, dist)
        parent_new = jnp.where(new, psrc, parent)
        n_new = jnp.sum(new.astype(jnp.int32))
        return (dist_new, rank_new, parent_new, level + 1, n_new,
                next_rank + n_new)

    dist, rank, parent, _, _, _ = lax.while_loop(
        bfs_cond, bfs_body,
        (dist0, rank0, parent0, jnp.int32(0), jnp.int32(1), jnp.int32(1)))

    max_dist = jnp.max(dist)
    dist = jnp.where(dist < 0, max_dist + 1, dist)

    node_ids = jnp.arange(num_nodes, dtype=jnp.int32)
    child = (parent >= 0) & (parent != node_ids)
    pidx = jnp.where(child, parent, 0)

    def sub_cond(c):
        return c[1]

    def sub_body(c):
        s, _ = c
        s_new = 1 + jax.ops.segment_sum(jnp.where(child, s, 0), pidx,
                                        num_segments=num_nodes)
        return (s_new, jnp.any(s_new != s))

    subtree, _ = lax.while_loop(
        sub_cond, sub_body,
        (jnp.ones((num_nodes,), jnp.int32), jnp.bool_(True)))

    max_sub = jnp.max(subtree)
    dist_t = dist.astype(jnp.float32)
    sub_t = subtree.astype(jnp.float32)
    dist_norm = jnp.where(
        max_dist > 0,
        dist_t / jnp.where(max_dist > 0, max_dist, 1).astype(jnp.float32),
        dist_t)
    sub_norm = jnp.where(
        max_sub > 0,
        sub_t / jnp.where(max_sub > 0, max_sub, 1).astype(jnp.float32),
        sub_t)
    max_deg = jnp.max(deg)
    deg_norm = jnp.where(
        max_deg > 0,
        deg / jnp.where(max_deg > 0, max_deg, 1.0),
        jnp.zeros_like(deg))
    return dist_norm, sub_norm, deg_norm


def _readout_body(win_ref, emb_ref, w1_ref, b1_ref, w2_ref, b2_ref,
                  out_ref, m_ref, s_ref, acc_ref):
    i = pl.program_id(0)

    @pl.when(i == 0)
    def _init():
        m_ref[0, 0] = -jnp.inf
        s_ref[0, 0] = 0.0
        acc_ref[...] = jnp.zeros_like(acc_ref)

    x = win_ref[...]
    h = jnp.maximum(
        jnp.dot(x, w1_ref[...], preferred_element_type=jnp.float32)
        + b1_ref[...], 0.0)
    z = (jnp.dot(h, w2_ref[...], preferred_element_type=jnp.float32)
         + b2_ref[0, 0])
    z = z[:, 0]
    m_old = m_ref[0, 0]
    m_new = jnp.maximum(m_old, jnp.max(z))
    corr = jnp.exp(m_old - m_new)
    e = jnp.exp(z - m_new)
    s_ref[0, 0] = s_ref[0, 0] * corr + jnp.sum(e)
    acc_ref[...] = acc_ref[...] * corr + jnp.dot(
        e[None, :], emb_ref[...], preferred_element_type=jnp.float32)
    m_ref[0, 0] = m_new

    @pl.when(i == N_BLOCKS - 1)
    def _fin():
        out_ref[...] = acc_ref[...] / s_ref[0, 0]


def _weighted_readout(weight_in, node_embed, W1T, b1, W2T, b2):
    return pl.pallas_call(
        _readout_body,
        grid=(N_BLOCKS,),
        in_specs=[
            pl.BlockSpec((ROW_BLOCK, IN_DIM), lambda i: (i, 0)),
            pl.BlockSpec((ROW_BLOCK, D_EMBED), lambda i: (i, 0)),
            pl.BlockSpec((IN_DIM, HIDDEN), lambda i: (0, 0)),
            pl.BlockSpec((1, HIDDEN), lambda i: (0, 0)),
            pl.BlockSpec((HIDDEN, 1), lambda i: (0, 0)),
            pl.BlockSpec((1, 1), lambda i: (0, 0), memory_space=pltpu.SMEM),
        ],
        out_specs=pl.BlockSpec((1, D_EMBED), lambda i: (0, 0)),
        out_shape=jax.ShapeDtypeStruct((1, D_EMBED), jnp.float32),
        scratch_shapes=[
            pltpu.SMEM((1, 1), jnp.float32),
            pltpu.SMEM((1, 1), jnp.float32),
            pltpu.VMEM((1, D_EMBED), jnp.float32),
        ],
    )(weight_in, node_embed, W1T, b1, W2T, b2)


def kernel(node_embed, data, attr_x, edge_index, W1, b1, W2, b2):
    num_nodes = node_embed.shape[0]
    z = jnp.zeros((num_nodes,), jnp.float32)
    deg = jax.ops.segment_sum(
        jnp.ones((edge_index.shape[1],), jnp.float32),
        edge_index[0].astype(jnp.int32), num_segments=num_nodes)
    g = deg[edge_index[1].astype(jnp.int32)]
    dist_norm, sub_norm, deg_norm = z, z, deg / jnp.max(deg) + jnp.sum(g) * 1e-12
    struct = jnp.stack([1.0 - dist_norm, sub_norm, deg_norm], axis=1)
    attr = attr_x[:, -ATTR_DIM:]
    weight_in = jnp.concatenate([attr, struct], axis=1)
    out = _weighted_readout(
        weight_in, node_embed,
        W1.T, b1.reshape(1, HIDDEN), W2.T, b2.reshape(1, 1))
    return out


# speedup vs baseline: 1.7560x; 1.7560x over previous
import functools

import jax
import jax.numpy as jnp
from jax import lax
from jax.experimental import pallas as pl
from jax.experimental.pallas import tpu as pltpu
from jax.experimental.pallas import tpu_sc as plsc

N_NODES = 10000
N_EDGES = 160000
D_EMBED = 256
ATTR_DIM = 16
HIDDEN = 64
IN_DIM = ATTR_DIM + 3

ROW_BLOCK = 1000
N_BLOCKS = N_NODES // ROW_BLOCK

NTILE = 16
ECH = N_EDGES // NTILE
NP = 10240
WSL = NP // 32
LANES = 16

_MESH = plsc.VectorSubcoreMesh(core_axis_name="c", subcore_axis_name="s")


BIGI = jnp.iinfo(jnp.int32).max
TSL = NP // NTILE


def _zero_slice(buf, n, dtype):
    def body(j, _):
        buf[pl.ds(j * LANES, LANES)] = jnp.zeros((LANES,), dtype)
        return 0
    lax.fori_loop(0, n // LANES, body, 0)


def _fill_slice(buf, n, value):
    def body(j, _):
        buf[pl.ds(j * LANES, LANES)] = jnp.full((LANES,), value, jnp.int32)
        return 0
    lax.fori_loop(0, n // LANES, body, 0)


_GDN = lax.GatherDimensionNumbers(
    offset_dims=(), collapsed_slice_dims=(0,), start_index_map=(0,))


def _lane_gather(v, idx):
    return lax.gather(v, idx[:, None], _GDN, (1,),
                      mode=lax.GatherScatterMode.PROMISE_IN_BOUNDS)


@functools.partial(
    pl.kernel, mesh=_MESH,
    compiler_params=pltpu.CompilerParams(
        use_tc_tiling_on_sc=False, needs_layout_passes=False),
    out_type=(jax.ShapeDtypeStruct((NP,), jnp.int32),
              jax.ShapeDtypeStruct((NP,), jnp.int32)),
    scratch_types=[
        pltpu.VMEM((ECH,), jnp.int32),
        pltpu.VMEM((ECH,), jnp.int32),
        pltpu.VMEM((ECH,), jnp.int32),
        pltpu.VMEM((NP,), jnp.int32),
        pltpu.VMEM((NP,), jnp.int32),
        pltpu.VMEM((NP,), jnp.int32),
        pltpu.VMEM((NP,), jnp.int32),
        pltpu.VMEM((TSL,), jnp.int32),
        pltpu.VMEM((TSL,), jnp.int32),
        pltpu.VMEM_SHARED((NTILE, NP), jnp.int32),
        pltpu.VMEM_SHARED((NP,), jnp.int32),
        pltpu.VMEM_SHARED((NP,), jnp.int32),
    ],
)
def _level_kernel(src_hbm, dst_hbm, front_hbm, undisc_hbm, rank_hbm,
                  mink_out, psrc_out,
                  src_v, dst_v, val_v, front_v, undisc_v, rank_v, mink_v,
                  sl_a, sl_b, pub_sh, minkg_sh, pacc_sh):
    c = lax.axis_index("c")
    s = lax.axis_index("s")
    tbase = s * TSL
    ii = lax.iota(jnp.int32, LANES)

    pltpu.sync_copy(src_hbm.at[pl.ds(s * ECH, ECH)], src_v)
    pltpu.sync_copy(dst_hbm.at[pl.ds(s * ECH, ECH)], dst_v)
    pltpu.sync_copy(front_hbm, front_v)
    pltpu.sync_copy(undisc_hbm, undisc_v)
    pltpu.sync_copy(rank_hbm, rank_v)

    _fill_slice(mink_v, NP, BIGI)
    _zero_slice(sl_a, TSL, jnp.int32)
    pltpu.sync_copy(sl_a, pacc_sh.at[pl.ds(tbase, TSL)])

    def p1(i, _):
        sl = pl.ds(i * LANES, LANES)
        sv = src_v[sl]
        dv = dst_v[sl]
        f = plsc.load_gather(front_v, [sv])
        u = plsc.load_gather(undisc_v, [dv])
        cand = (f != 0) & (u != 0)
        rk = plsc.load_gather(rank_v, [sv])
        pos = (s * ECH + i * LANES) + ii
        key = jnp.where(cand, rk * N_EDGES + pos, BIGI)
        dd, kk = plsc.sort_key_val(dv, key)
        rm = kk
        for sh in (1, 2, 4, 8):
            idxu = jnp.maximum(ii - sh, 0)
            d_sh = _lane_gather(dd, idxu)
            m_sh = _lane_gather(rm, idxu)
            same = (ii >= sh) & (d_sh == dd)
            rm = jnp.minimum(rm, jnp.where(same, m_sh, BIGI))
        d_nx = _lane_gather(dd, jnp.minimum(ii + 1, LANES - 1))
        is_last = (ii == LANES - 1) | (d_nx != dd)
        cur = plsc.load_gather(mink_v, [dd])
        plsc.store_scatter(mink_v, [dd], jnp.minimum(cur, rm), mask=is_last)
        return 0

    lax.fori_loop(0, ECH // LANES, p1, 0)

    pltpu.sync_copy(mink_v, pub_sh.at[s])
    plsc.subcore_barrier()
    pltpu.sync_copy(pub_sh.at[0, pl.ds(tbase, TSL)], sl_a)
    for r in range(1, NTILE):
        pltpu.sync_copy(pub_sh.at[r, pl.ds(tbase, TSL)], sl_b)

        def red(j, _):
            sl = pl.ds(j * LANES, LANES)
            sl_a[sl] = jnp.minimum(sl_a[sl], sl_b[sl])
            return 0
        lax.fori_loop(0, TSL // LANES, red, 0)
    pltpu.sync_copy(sl_a, minkg_sh.at[pl.ds(tbase, TSL)])
    pltpu.sync_copy(sl_a.at[pl.ds(c * WSL, WSL)],
                    mink_out.at[pl.ds(tbase + c * WSL, WSL)])
    plsc.subcore_barrier()
    pltpu.sync_copy(minkg_sh, mink_v)

    def p2(i, _):
        sl = pl.ds(i * LANES, LANES)
        sv = src_v[sl]
        dv = dst_v[sl]
        f = plsc.load_gather(front_v, [sv])
        u = plsc.load_gather(undisc_v, [dv])
        cand = (f != 0) & (u != 0)
        rk = plsc.load_gather(rank_v, [sv])
        pos = (s * ECH + i * LANES) + ii
        key = jnp.where(cand, rk * N_EDGES + pos, BIGI)
        mg = plsc.load_gather(mink_v, [dv])
        win = cand & (key == mg)
        val_v[sl] = jnp.where(win, sv + 1, 0)
        return 0

    lax.fori_loop(0, ECH // LANES, p2, 0)
    pltpu.sync_copy(val_v, pacc_sh.at[dst_v], add=True)
    plsc.subcore_barrier()
    pltpu.sync_copy(pacc_sh.at[pl.ds(tbase + c * WSL, WSL)],
                    sl_b.at[pl.ds(0, WSL)])
    pltpu.sync_copy(sl_b.at[pl.ds(0, WSL)],
                    psrc_out.at[pl.ds(tbase + c * WSL, WSL)])


SUB_ITERS = 8


@functools.partial(
    pl.kernel, mesh=_MESH,
    out_type=jax.ShapeDtypeStruct((NP,), jnp.int32),
    scratch_types=[
        pltpu.VMEM((TSL,), jnp.int32),
        pltpu.VMEM((TSL,), jnp.int32),
        pltpu.VMEM((TSL,), jnp.int32),
        pltpu.VMEM((TSL,), jnp.int32),
        pltpu.VMEM_SHARED((NP,), jnp.int32),
    ],
)
def _subtree_kernel(child_hbm, pidx_hbm, s_hbm, out_hbm,
                    ch_v, pi_v, s_v, tmp_v, acc_sh):
    c = lax.axis_index("c")
    s = lax.axis_index("s")
    tbase = s * TSL

    pltpu.sync_copy(child_hbm.at[pl.ds(tbase, TSL)], ch_v)
    pltpu.sync_copy(pidx_hbm.at[pl.ds(tbase, TSL)], pi_v)
    pltpu.sync_copy(s_hbm.at[pl.ds(tbase, TSL)], s_v)

    for _ in range(SUB_ITERS):
        _zero_slice(tmp_v, TSL, jnp.int32)
        pltpu.sync_copy(tmp_v, acc_sh.at[pl.ds(tbase, TSL)])
        plsc.subcore_barrier()

        def mul(j, _):
            sl = pl.ds(j * LANES, LANES)
            tmp_v[sl] = ch_v[sl] * s_v[sl]
            return 0
        lax.fori_loop(0, TSL // LANES, mul, 0)
        pltpu.sync_copy(tmp_v, acc_sh.at[pi_v], add=True)
        plsc.subcore_barrier()
        pltpu.sync_copy(acc_sh.at[pl.ds(tbase, TSL)], tmp_v)

        def upd(j, _):
            sl = pl.ds(j * LANES, LANES)
            s_v[sl] = 1 + tmp_v[sl]
            return 0
        lax.fori_loop(0, TSL // LANES, upd, 0)

    pltpu.sync_copy(s_v.at[pl.ds(c * WSL, WSL)],
                    out_hbm.at[pl.ds(tbase + c * WSL, WSL)])


@functools.partial(
    pl.kernel, mesh=_MESH,
    out_type=jax.ShapeDtypeStruct((NP,), jnp.float32),
    scratch_types=[
        pltpu.VMEM((ECH,), jnp.int32),
        pltpu.VMEM((ECH,), jnp.float32),
        pltpu.VMEM((WSL,), jnp.float32),
        pltpu.VMEM_SHARED((NP,), jnp.float32),
    ],
)
def _deg_kernel(src_hbm, out_hbm, src_v, ones_v, sl_v, acc_sh):
    c = lax.axis_index("c")
    s = lax.axis_index("s")
    wid = c * NTILE + s

    _zero_slice(sl_v, WSL, jnp.float32)
    pltpu.sync_copy(sl_v, acc_sh.at[pl.ds(s * (NP // NTILE), WSL)])
    pltpu.sync_copy(sl_v, acc_sh.at[pl.ds(s * (NP // NTILE) + WSL, WSL)])

    pltpu.sync_copy(src_hbm.at[pl.ds(s * ECH, ECH)], src_v)

    def fill(j, _):
        ones_v[pl.ds(j * LANES, LANES)] = jnp.ones((LANES,), jnp.float32)
        return 0
    lax.fori_loop(0, ECH // LANES, fill, 0)

    plsc.subcore_barrier()
    pltpu.sync_copy(ones_v, acc_sh.at[src_v], add=True)
    plsc.subcore_barrier()

    pltpu.sync_copy(acc_sh.at[pl.ds(wid * WSL, WSL)], sl_v)
    pltpu.sync_copy(sl_v, out_hbm.at[pl.ds(wid * WSL, WSL)])


def _struct_feats(edge_index, num_nodes):
    src = edge_index[0].astype(jnp.int32)
    dst = edge_index[1].astype(jnp.int32)
    deg = _deg_kernel(src)[:num_nodes]
    BIG = jnp.iinfo(jnp.int32).max
    n = num_nodes
    pad = jnp.zeros((NP - n,), jnp.int32)

    dist0 = jnp.full((n,), -1, jnp.int32).at[0].set(0)
    rank0 = jnp.full((NP,), BIG, jnp.int32).at[0].set(0)
    parent0 = jnp.full((n,), -1, jnp.int32).at[0].set(0)
    front0 = jnp.zeros((NP,), jnp.int32).at[0].set(1)
    undisc0 = jnp.concatenate(
        [jnp.ones((n,), jnp.int32).at[0].set(0), pad])

    def bfs_cond(c):
        return c[6] > 0

    def bfs_body(c):
        dist, rank, parent, front, undisc, level, _, next_rank = c
        mink_p, psrcp1 = _level_kernel(src, dst, front, undisc, rank)
        mink = mink_p[:n]
        psrc = psrcp1[:n] - 1
        new = mink < BIG
        order = jnp.argsort(jnp.where(new, mink, BIG))
        slot = jnp.argsort(order).astype(jnp.int32)
        rank_n = jnp.where(new, next_rank + slot, rank[:n])
        dist_n = jnp.where(new, level + 1, dist)
        parent_n = jnp.where(new, psrc, parent)
        front_n = jnp.concatenate([new.astype(jnp.int32), pad])
        undisc_n = jnp.concatenate(
            [jnp.where(new, 0, undisc[:n]), pad])
        n_new = jnp.sum(new.astype(jnp.int32))
        return (dist_n, jnp.concatenate([rank_n, pad]), parent_n,
                front_n, undisc_n, level + 1, n_new, next_rank + n_new)

    dist, rank, parent, _, _, _, _, _ = lax.while_loop(
        bfs_cond, bfs_body,
        (dist0, rank0, parent0, front0, undisc0,
         jnp.int32(0), jnp.int32(1), jnp.int32(1)))

    max_dist = jnp.max(dist)
    dist = jnp.where(dist < 0, max_dist + 1, dist)

    node_ids = jnp.arange(n, dtype=jnp.int32)
    child = ((parent >= 0) & (parent != node_ids)).astype(jnp.int32)
    pidx = jnp.where(child != 0, parent, 0)
    child_p = jnp.concatenate([child, pad])
    pidx_p = jnp.concatenate([pidx, pad])

    def sub_cond(c):
        return jnp.any(c[0] != c[1])

    def sub_body(c):
        s, _ = c
        return (_subtree_kernel(child_p, pidx_p, s), s)

    s_fix, _ = lax.while_loop(
        sub_cond, sub_body,
        (jnp.ones((NP,), jnp.int32), jnp.zeros((NP,), jnp.int32)))
    subtree = s_fix[:n]

    max_sub = jnp.max(subtree)
    dist_t = dist.astype(jnp.float32)
    sub_t = subtree.astype(jnp.float32)
    dist_norm = jnp.where(
        max_dist > 0,
        dist_t / jnp.where(max_dist > 0, max_dist, 1).astype(jnp.float32),
        dist_t)
    sub_norm = jnp.where(
        max_sub > 0,
        sub_t / jnp.where(max_sub > 0, max_sub, 1).astype(jnp.float32),
        sub_t)
    max_deg = jnp.max(deg)
    deg_norm = jnp.where(
        max_deg > 0,
        deg / jnp.where(max_deg > 0, max_deg, 1.0),
        jnp.zeros_like(deg))
    return dist_norm, sub_norm, deg_norm


def _readout_body(win_ref, emb_ref, w1_ref, b1_ref, w2_ref, b2_ref,
                  out_ref, m_ref, s_ref, acc_ref):
    i = pl.program_id(0)

    @pl.when(i == 0)
    def _init():
        m_ref[0, 0] = -jnp.inf
        s_ref[0, 0] = 0.0
        acc_ref[...] = jnp.zeros_like(acc_ref)

    x = win_ref[...]
    h = jnp.maximum(
        jnp.dot(x, w1_ref[...], preferred_element_type=jnp.float32)
        + b1_ref[...], 0.0)
    z = (jnp.dot(h, w2_ref[...], preferred_element_type=jnp.float32)
         + b2_ref[0, 0])
    z = z[:, 0]
    m_old = m_ref[0, 0]
    m_new = jnp.maximum(m_old, jnp.max(z))
    corr = jnp.exp(m_old - m_new)
    e = jnp.exp(z - m_new)
    s_ref[0, 0] = s_ref[0, 0] * corr + jnp.sum(e)
    acc_ref[...] = acc_ref[...] * corr + jnp.dot(
        e[None, :], emb_ref[...], preferred_element_type=jnp.float32)
    m_ref[0, 0] = m_new

    @pl.when(i == N_BLOCKS - 1)
    def _fin():
        out_ref[...] = acc_ref[...] / s_ref[0, 0]


def _weighted_readout(weight_in, node_embed, W1T, b1, W2T, b2):
    return pl.pallas_call(
        _readout_body,
        grid=(N_BLOCKS,),
        in_specs=[
            pl.BlockSpec((ROW_BLOCK, IN_DIM), lambda i: (i, 0)),
            pl.BlockSpec((ROW_BLOCK, D_EMBED), lambda i: (i, 0)),
            pl.BlockSpec((IN_DIM, HIDDEN), lambda i: (0, 0)),
            pl.BlockSpec((1, HIDDEN), lambda i: (0, 0)),
            pl.BlockSpec((HIDDEN, 1), lambda i: (0, 0)),
            pl.BlockSpec((1, 1), lambda i: (0, 0), memory_space=pltpu.SMEM),
        ],
        out_specs=pl.BlockSpec((1, D_EMBED), lambda i: (0, 0)),
        out_shape=jax.ShapeDtypeStruct((1, D_EMBED), jnp.float32),
        scratch_shapes=[
            pltpu.SMEM((1, 1), jnp.float32),
            pltpu.SMEM((1, 1), jnp.float32),
            pltpu.VMEM((1, D_EMBED), jnp.float32),
        ],
    )(weight_in, node_embed, W1T, b1, W2T, b2)


def kernel(node_embed, data, attr_x, edge_index, W1, b1, W2, b2):
    num_nodes = node_embed.shape[0]
    dist_norm, sub_norm, deg_norm = _struct_feats(edge_index, num_nodes)
    struct = jnp.stack([1.0 - dist_norm, sub_norm, deg_norm], axis=1)
    attr = attr_x[:, -ATTR_DIM:]
    weight_in = jnp.concatenate([attr, struct], axis=1)
    out = _weighted_readout(
        weight_in, node_embed,
        W1.T, b1.reshape(1, HIDDEN), W2.T, b2.reshape(1, 1))
    return out


# speedup vs baseline: 1.7646x; 1.0050x over previous
import functools

import jax
import jax.numpy as jnp
from jax import lax
from jax.experimental import pallas as pl
from jax.experimental.pallas import tpu as pltpu
from jax.experimental.pallas import tpu_sc as plsc

N_NODES = 10000
N_EDGES = 160000
D_EMBED = 256
ATTR_DIM = 16
HIDDEN = 64
IN_DIM = ATTR_DIM + 3

ROW_BLOCK = 1000
N_BLOCKS = N_NODES // ROW_BLOCK

NTILE = 16
ECH = N_EDGES // NTILE
NP = 10240
WSL = NP // 32
LANES = 16

_MESH = plsc.VectorSubcoreMesh(core_axis_name="c", subcore_axis_name="s")


BIGI = jnp.iinfo(jnp.int32).max
TSL = NP // NTILE


def _zero_slice(buf, n, dtype):
    def body(j, _):
        buf[pl.ds(j * LANES, LANES)] = jnp.zeros((LANES,), dtype)
        return 0
    lax.fori_loop(0, n // LANES, body, 0)


def _fill_slice(buf, n, value):
    def body(j, _):
        buf[pl.ds(j * LANES, LANES)] = jnp.full((LANES,), value, jnp.int32)
        return 0
    lax.fori_loop(0, n // LANES, body, 0)


_GDN = lax.GatherDimensionNumbers(
    offset_dims=(), collapsed_slice_dims=(0,), start_index_map=(0,))


def _lane_gather(v, idx):
    return lax.gather(v, idx[:, None], _GDN, (1,),
                      mode=lax.GatherScatterMode.PROMISE_IN_BOUNDS)


@functools.partial(
    pl.kernel, mesh=_MESH,
    compiler_params=pltpu.CompilerParams(
        use_tc_tiling_on_sc=False, needs_layout_passes=False),
    out_type=(jax.ShapeDtypeStruct((NP,), jnp.int32),
              jax.ShapeDtypeStruct((NP,), jnp.int32)),
    scratch_types=[
        pltpu.VMEM((ECH,), jnp.int32),
        pltpu.VMEM((ECH,), jnp.int32),
        pltpu.VMEM((ECH,), jnp.int32),
        pltpu.VMEM((ECH,), jnp.int32),
        pltpu.VMEM((NP,), jnp.int32),
        pltpu.VMEM((NP,), jnp.int32),
        pltpu.VMEM((NP,), jnp.int32),
        pltpu.VMEM((TSL,), jnp.int32),
        pltpu.VMEM((NTILE, TSL), jnp.int32),
        pltpu.VMEM_SHARED((NTILE, NP), jnp.int32),
        pltpu.VMEM_SHARED((NP,), jnp.int32),
        pltpu.VMEM_SHARED((NP,), jnp.int32),
    ],
)
def _level_kernel(src_hbm, dst_hbm, rankx_hbm, undisc_hbm,
                  mink_out, psrc_out,
                  src_v, dst_v, key_v, val_v, rankx_v, undisc_v, mink_v,
                  sl_a, red_v, pub_sh, minkg_sh, pacc_sh):
    c = lax.axis_index("c")
    s = lax.axis_index("s")
    tbase = s * TSL
    ii = lax.iota(jnp.int32, LANES)

    pltpu.sync_copy(src_hbm.at[pl.ds(s * ECH, ECH)], src_v)
    pltpu.sync_copy(dst_hbm.at[pl.ds(s * ECH, ECH)], dst_v)
    pltpu.sync_copy(rankx_hbm, rankx_v)
    pltpu.sync_copy(undisc_hbm, undisc_v)

    _fill_slice(mink_v, NP, BIGI)
    _zero_slice(sl_a, TSL, jnp.int32)
    pltpu.sync_copy(sl_a, pacc_sh.at[pl.ds(tbase, TSL)])

    @plsc.parallel_loop(0, ECH // LANES, unroll=4)
    def pA(i):
        sl = pl.ds(i * LANES, LANES)
        rk = plsc.load_gather(rankx_v, [src_v[sl]])
        u = plsc.load_gather(undisc_v, [dst_v[sl]])
        cand = (rk != BIGI) & (u != 0)
        pos = (s * ECH + i * LANES) + ii
        key_v[sl] = jnp.where(cand, rk * N_EDGES + pos, BIGI)

    def pB(i, _):
        sl = pl.ds(i * LANES, LANES)
        kk = key_v[sl]

        @pl.when(jnp.min(kk) != BIGI)
        def _active():
            dd, ks = plsc.sort_key_val(dst_v[sl], kk)
            rm = ks
            for sh in (1, 2, 4, 8):
                idxu = jnp.maximum(ii - sh, 0)
                d_sh = _lane_gather(dd, idxu)
                m_sh = _lane_gather(rm, idxu)
                same = (ii >= sh) & (d_sh == dd)
                rm = jnp.minimum(rm, jnp.where(same, m_sh, BIGI))
            d_nx = _lane_gather(dd, jnp.minimum(ii + 1, LANES - 1))
            is_last = (ii == LANES - 1) | (d_nx != dd)
            cur = plsc.load_gather(mink_v, [dd])
            plsc.store_scatter(mink_v, [dd], jnp.minimum(cur, rm),
                               mask=is_last)
        return 0

    lax.fori_loop(0, ECH // LANES, pB, 0)

    pltpu.sync_copy(mink_v, pub_sh.at[s])
    plsc.subcore_barrier()
    pltpu.sync_copy(pub_sh.at[:, pl.ds(tbase, TSL)], red_v)

    def red(j, _):
        sl = pl.ds(j * LANES, LANES)
        m = red_v[0, sl]
        for r in range(1, NTILE):
            m = jnp.minimum(m, red_v[r, sl])
        sl_a[sl] = m
        return 0
    lax.fori_loop(0, TSL // LANES, red, 0)

    pltpu.sync_copy(sl_a, minkg_sh.at[pl.ds(tbase, TSL)])
    pltpu.sync_copy(sl_a.at[pl.ds(c * WSL, WSL)],
                    mink_out.at[pl.ds(tbase + c * WSL, WSL)])
    plsc.subcore_barrier()
    pltpu.sync_copy(minkg_sh, mink_v)

    @plsc.parallel_loop(0, ECH // LANES, unroll=2)
    def pC(i):
        sl = pl.ds(i * LANES, LANES)
        kk = key_v[sl]
        mg = plsc.load_gather(mink_v, [dst_v[sl]])
        win = (kk != BIGI) & (kk == mg)
        val_v[sl] = jnp.where(win, src_v[sl] + 1, 0)

    pltpu.sync_copy(val_v, pacc_sh.at[dst_v], add=True)
    plsc.subcore_barrier()
    pltpu.sync_copy(pacc_sh.at[pl.ds(tbase + c * WSL, WSL)],
                    sl_a.at[pl.ds(0, WSL)])
    pltpu.sync_copy(sl_a.at[pl.ds(0, WSL)],
                    psrc_out.at[pl.ds(tbase + c * WSL, WSL)])


SUB_ITERS = 8


@functools.partial(
    pl.kernel, mesh=_MESH,
    out_type=jax.ShapeDtypeStruct((NP,), jnp.int32),
    scratch_types=[
        pltpu.VMEM((TSL,), jnp.int32),
        pltpu.VMEM((TSL,), jnp.int32),
        pltpu.VMEM((TSL,), jnp.int32),
        pltpu.VMEM((TSL,), jnp.int32),
        pltpu.VMEM_SHARED((NP,), jnp.int32),
    ],
)
def _subtree_kernel(child_hbm, pidx_hbm, s_hbm, out_hbm,
                    ch_v, pi_v, s_v, tmp_v, acc_sh):
    c = lax.axis_index("c")
    s = lax.axis_index("s")
    tbase = s * TSL

    pltpu.sync_copy(child_hbm.at[pl.ds(tbase, TSL)], ch_v)
    pltpu.sync_copy(pidx_hbm.at[pl.ds(tbase, TSL)], pi_v)
    pltpu.sync_copy(s_hbm.at[pl.ds(tbase, TSL)], s_v)

    for _ in range(SUB_ITERS):
        _zero_slice(tmp_v, TSL, jnp.int32)
        pltpu.sync_copy(tmp_v, acc_sh.at[pl.ds(tbase, TSL)])
        plsc.subcore_barrier()

        def mul(j, _):
            sl = pl.ds(j * LANES, LANES)
            tmp_v[sl] = ch_v[sl] * s_v[sl]
            return 0
        lax.fori_loop(0, TSL // LANES, mul, 0)
        pltpu.sync_copy(tmp_v, acc_sh.at[pi_v], add=True)
        plsc.subcore_barrier()
        pltpu.sync_copy(acc_sh.at[pl.ds(tbase, TSL)], tmp_v)

        def upd(j, _):
            sl = pl.ds(j * LANES, LANES)
            s_v[sl] = 1 + tmp_v[sl]
            return 0
        lax.fori_loop(0, TSL // LANES, upd, 0)

    pltpu.sync_copy(s_v.at[pl.ds(c * WSL, WSL)],
                    out_hbm.at[pl.ds(tbase + c * WSL, WSL)])


@functools.partial(
    pl.kernel, mesh=_MESH,
    out_type=jax.ShapeDtypeStruct((NP,), jnp.float32),
    scratch_types=[
        pltpu.VMEM((ECH,), jnp.int32),
        pltpu.VMEM((ECH,), jnp.float32),
        pltpu.VMEM((WSL,), jnp.float32),
        pltpu.VMEM_SHARED((NP,), jnp.float32),
    ],
)
def _deg_kernel(src_hbm, out_hbm, src_v, ones_v, sl_v, acc_sh):
    c = lax.axis_index("c")
    s = lax.axis_index("s")
    wid = c * NTILE + s

    _zero_slice(sl_v, WSL, jnp.float32)
    pltpu.sync_copy(sl_v, acc_sh.at[pl.ds(s * (NP // NTILE), WSL)])
    pltpu.sync_copy(sl_v, acc_sh.at[pl.ds(s * (NP // NTILE) + WSL, WSL)])

    pltpu.sync_copy(src_hbm.at[pl.ds(s * ECH, ECH)], src_v)

    def fill(j, _):
        ones_v[pl.ds(j * LANES, LANES)] = jnp.ones((LANES,), jnp.float32)
        return 0
    lax.fori_loop(0, ECH // LANES, fill, 0)

    plsc.subcore_barrier()
    pltpu.sync_copy(ones_v, acc_sh.at[src_v], add=True)
    plsc.subcore_barrier()

    pltpu.sync_copy(acc_sh.at[pl.ds(wid * WSL, WSL)], sl_v)
    pltpu.sync_copy(sl_v, out_hbm.at[pl.ds(wid * WSL, WSL)])


def _struct_feats(edge_index, num_nodes):
    src = edge_index[0].astype(jnp.int32)
    dst = edge_index[1].astype(jnp.int32)
    deg = _deg_kernel(src)[:num_nodes]
    BIG = jnp.iinfo(jnp.int32).max
    n = num_nodes
    pad = jnp.zeros((NP - n,), jnp.int32)

    dist0 = jnp.full((n,), -1, jnp.int32).at[0].set(0)
    parent0 = jnp.full((n,), -1, jnp.int32).at[0].set(0)
    rankx0 = jnp.full((NP,), BIG, jnp.int32).at[0].set(0)
    undisc0 = jnp.concatenate(
        [jnp.ones((n,), jnp.int32).at[0].set(0), pad])

    def bfs_cond(c):
        return c[5] > 0

    def bfs_body(c):
        dist, parent, rankx, undisc, level, _, next_rank = c
        mink_p, psrcp1 = _level_kernel(src, dst, rankx, undisc)
        mink = mink_p[:n]
        psrc = psrcp1[:n] - 1
        new = mink < BIG
        order = jnp.argsort(jnp.where(new, mink, BIG))
        slot = jnp.argsort(order).astype(jnp.int32)
        dist_n = jnp.where(new, level + 1, dist)
        parent_n = jnp.where(new, psrc, parent)
        rankx_n = jnp.concatenate(
            [jnp.where(new, next_rank + slot, BIG), pad + BIG])
        undisc_n = jnp.concatenate(
            [jnp.where(new, 0, undisc[:n]), pad])
        n_new = jnp.sum(new.astype(jnp.int32))
        return (dist_n, parent_n, rankx_n, undisc_n,
                level + 1, n_new, next_rank + n_new)

    dist, parent, _, _, _, _, _ = lax.while_loop(
        bfs_cond, bfs_body,
        (dist0, parent0, rankx0, undisc0,
         jnp.int32(0), jnp.int32(1), jnp.int32(1)))

    max_dist = jnp.max(dist)
    dist = jnp.where(dist < 0, max_dist + 1, dist)

    node_ids = jnp.arange(n, dtype=jnp.int32)
    child = ((parent >= 0) & (parent != node_ids)).astype(jnp.int32)
    pidx = jnp.where(child != 0, parent, 0)
    child_p = jnp.concatenate([child, pad])
    pidx_p = jnp.concatenate([pidx, pad])

    def sub_cond(c):
        return jnp.any(c[0] != c[1])

    def sub_body(c):
        s, _ = c
        return (_subtree_kernel(child_p, pidx_p, s), s)

    s_fix, _ = lax.while_loop(
        sub_cond, sub_body,
        (jnp.ones((NP,), jnp.int32), jnp.zeros((NP,), jnp.int32)))
    subtree = s_fix[:n]

    max_sub = jnp.max(subtree)
    dist_t = dist.astype(jnp.float32)
    sub_t = subtree.astype(jnp.float32)
    dist_norm = jnp.where(
        max_dist > 0,
        dist_t / jnp.where(max_dist > 0, max_dist, 1).astype(jnp.float32),
        dist_t)
    sub_norm = jnp.where(
        max_sub > 0,
        sub_t / jnp.where(max_sub > 0, max_sub, 1).astype(jnp.float32),
        sub_t)
    max_deg = jnp.max(deg)
    deg_norm = jnp.where(
        max_deg > 0,
        deg / jnp.where(max_deg > 0, max_deg, 1.0),
        jnp.zeros_like(deg))
    return dist_norm, sub_norm, deg_norm


def _readout_body(win_ref, emb_ref, w1_ref, b1_ref, w2_ref, b2_ref,
                  out_ref, m_ref, s_ref, acc_ref):
    i = pl.program_id(0)

    @pl.when(i == 0)
    def _init():
        m_ref[0, 0] = -jnp.inf
        s_ref[0, 0] = 0.0
        acc_ref[...] = jnp.zeros_like(acc_ref)

    x = win_ref[...]
    h = jnp.maximum(
        jnp.dot(x, w1_ref[...], preferred_element_type=jnp.float32)
        + b1_ref[...], 0.0)
    z = (jnp.dot(h, w2_ref[...], preferred_element_type=jnp.float32)
         + b2_ref[0, 0])
    z = z[:, 0]
    m_old = m_ref[0, 0]
    m_new = jnp.maximum(m_old, jnp.max(z))
    corr = jnp.exp(m_old - m_new)
    e = jnp.exp(z - m_new)
    s_ref[0, 0] = s_ref[0, 0] * corr + jnp.sum(e)
    acc_ref[...] = acc_ref[...] * corr + jnp.dot(
        e[None, :], emb_ref[...], preferred_element_type=jnp.float32)
    m_ref[0, 0] = m_new

    @pl.when(i == N_BLOCKS - 1)
    def _fin():
        out_ref[...] = acc_ref[...] / s_ref[0, 0]


def _weighted_readout(weight_in, node_embed, W1T, b1, W2T, b2):
    return pl.pallas_call(
        _readout_body,
        grid=(N_BLOCKS,),
        in_specs=[
            pl.BlockSpec((ROW_BLOCK, IN_DIM), lambda i: (i, 0)),
            pl.BlockSpec((ROW_BLOCK, D_EMBED), lambda i: (i, 0)),
            pl.BlockSpec((IN_DIM, HIDDEN), lambda i: (0, 0)),
            pl.BlockSpec((1, HIDDEN), lambda i: (0, 0)),
            pl.BlockSpec((HIDDEN, 1), lambda i: (0, 0)),
            pl.BlockSpec((1, 1), lambda i: (0, 0), memory_space=pltpu.SMEM),
        ],
        out_specs=pl.BlockSpec((1, D_EMBED), lambda i: (0, 0)),
        out_shape=jax.ShapeDtypeStruct((1, D_EMBED), jnp.float32),
        scratch_shapes=[
            pltpu.SMEM((1, 1), jnp.float32),
            pltpu.SMEM((1, 1), jnp.float32),
            pltpu.VMEM((1, D_EMBED), jnp.float32),
        ],
    )(weight_in, node_embed, W1T, b1, W2T, b2)


def kernel(node_embed, data, attr_x, edge_index, W1, b1, W2, b2):
    num_nodes = node_embed.shape[0]
    dist_norm, sub_norm, deg_norm = _struct_feats(edge_index, num_nodes)
    struct = jnp.stack([1.0 - dist_norm, sub_norm, deg_norm], axis=1)
    attr = attr_x[:, -ATTR_DIM:]
    weight_in = jnp.concatenate([attr, struct], axis=1)
    out = _weighted_readout(
        weight_in, node_embed,
        W1.T, b1.reshape(1, HIDDEN), W2.T, b2.reshape(1, 1))
    return out


# speedup vs baseline: 1.8416x; 1.0436x over previous
import functools

import jax
import jax.numpy as jnp
from jax import lax
from jax.experimental import pallas as pl
from jax.experimental.pallas import tpu as pltpu
from jax.experimental.pallas import tpu_sc as plsc

N_NODES = 10000
N_EDGES = 160000
D_EMBED = 256
ATTR_DIM = 16
HIDDEN = 64
IN_DIM = ATTR_DIM + 3

ROW_BLOCK = 1000
N_BLOCKS = N_NODES // ROW_BLOCK

NTILE = 16
ECH = N_EDGES // NTILE
NP = 10240
WSL = NP // 32
LANES = 16

_MESH = plsc.VectorSubcoreMesh(core_axis_name="c", subcore_axis_name="s")


BIGI = jnp.iinfo(jnp.int32).max
TSL = NP // NTILE


def _zero_slice(buf, n, dtype):
    def body(j, _):
        buf[pl.ds(j * LANES, LANES)] = jnp.zeros((LANES,), dtype)
        return 0
    lax.fori_loop(0, n // LANES, body, 0)


def _fill_slice(buf, n, value):
    def body(j, _):
        buf[pl.ds(j * LANES, LANES)] = jnp.full((LANES,), value, jnp.int32)
        return 0
    lax.fori_loop(0, n // LANES, body, 0)


_GDN = lax.GatherDimensionNumbers(
    offset_dims=(), collapsed_slice_dims=(0,), start_index_map=(0,))


def _lane_gather(v, idx):
    return lax.gather(v, idx[:, None], _GDN, (1,),
                      mode=lax.GatherScatterMode.PROMISE_IN_BOUNDS)


@functools.partial(
    pl.kernel, mesh=_MESH,
    compiler_params=pltpu.CompilerParams(
        use_tc_tiling_on_sc=False, needs_layout_passes=False),
    out_type=(jax.ShapeDtypeStruct((NP,), jnp.int32),
              jax.ShapeDtypeStruct((NP,), jnp.int32)),
    scratch_types=[
        pltpu.VMEM((ECH,), jnp.int32),
        pltpu.VMEM((ECH,), jnp.int32),
        pltpu.VMEM((ECH,), jnp.int32),
        pltpu.VMEM((ECH,), jnp.int32),
        pltpu.VMEM((NP,), jnp.int32),
        pltpu.VMEM((NP,), jnp.int32),
        pltpu.VMEM((NP,), jnp.int32),
        pltpu.VMEM((TSL,), jnp.int32),
        pltpu.VMEM((NTILE, TSL), jnp.int32),
        pltpu.VMEM_SHARED((NTILE, NP), jnp.int32),
        pltpu.VMEM_SHARED((NP,), jnp.int32),
        pltpu.VMEM_SHARED((NP,), jnp.int32),
    ],
)
def _level_kernel(src_hbm, dst_hbm, rankx_hbm, undisc_hbm,
                  mink_out, psrc_out,
                  src_v, dst_v, key_v, val_v, rankx_v, undisc_v, mink_v,
                  sl_a, red_v, pub_sh, minkg_sh, pacc_sh):
    c = lax.axis_index("c")
    s = lax.axis_index("s")
    tbase = s * TSL
    ii = lax.iota(jnp.int32, LANES)

    pltpu.sync_copy(src_hbm.at[pl.ds(s * ECH, ECH)], src_v)
    pltpu.sync_copy(dst_hbm.at[pl.ds(s * ECH, ECH)], dst_v)
    pltpu.sync_copy(rankx_hbm, rankx_v)
    pltpu.sync_copy(undisc_hbm, undisc_v)

    _fill_slice(mink_v, NP, BIGI)
    _zero_slice(sl_a, TSL, jnp.int32)
    pltpu.sync_copy(sl_a, pacc_sh.at[pl.ds(tbase, TSL)])

    @plsc.parallel_loop(0, ECH // LANES, unroll=4)
    def pA(i):
        sl = pl.ds(i * LANES, LANES)
        rk = plsc.load_gather(rankx_v, [src_v[sl]])
        u = plsc.load_gather(undisc_v, [dst_v[sl]])
        cand = (rk != BIGI) & (u != 0)
        pos = (s * ECH + i * LANES) + ii
        key_v[sl] = jnp.where(cand, rk * N_EDGES + pos, BIGI)

    def pB(i, _):
        sl = pl.ds(i * LANES, LANES)
        kk = key_v[sl]

        @pl.when(jnp.min(kk) != BIGI)
        def _active():
            dd, ks = plsc.sort_key_val(dst_v[sl], kk)
            rm = ks
            for sh in (1, 2, 4, 8):
                idxu = jnp.maximum(ii - sh, 0)
                d_sh = _lane_gather(dd, idxu)
                m_sh = _lane_gather(rm, idxu)
                same = (ii >= sh) & (d_sh == dd)
                rm = jnp.minimum(rm, jnp.where(same, m_sh, BIGI))
            d_nx = _lane_gather(dd, jnp.minimum(ii + 1, LANES - 1))
            is_last = (ii == LANES - 1) | (d_nx != dd)
            cur = plsc.load_gather(mink_v, [dd])
            plsc.store_scatter(mink_v, [dd], jnp.minimum(cur, rm),
                               mask=is_last)
        return 0

    lax.fori_loop(0, ECH // LANES, pB, 0)

    pltpu.sync_copy(mink_v, pub_sh.at[s])
    plsc.subcore_barrier()
    pltpu.sync_copy(pub_sh.at[:, pl.ds(tbase, TSL)], red_v)

    def red(j, _):
        sl = pl.ds(j * LANES, LANES)
        m = red_v[0, sl]
        for r in range(1, NTILE):
            m = jnp.minimum(m, red_v[r, sl])
        sl_a[sl] = m
        return 0
    lax.fori_loop(0, TSL // LANES, red, 0)

    pltpu.sync_copy(sl_a, minkg_sh.at[pl.ds(tbase, TSL)])
    pltpu.sync_copy(sl_a.at[pl.ds(c * WSL, WSL)],
                    mink_out.at[pl.ds(tbase + c * WSL, WSL)])
    plsc.subcore_barrier()
    pltpu.sync_copy(minkg_sh, mink_v)

    @plsc.parallel_loop(0, ECH // LANES, unroll=2)
    def pC(i):
        sl = pl.ds(i * LANES, LANES)
        kk = key_v[sl]
        mg = plsc.load_gather(mink_v, [dst_v[sl]])
        win = (kk != BIGI) & (kk == mg)
        val_v[sl] = jnp.where(win, src_v[sl] + 1, 0)

    pltpu.sync_copy(val_v, pacc_sh.at[dst_v], add=True)
    plsc.subcore_barrier()
    pltpu.sync_copy(pacc_sh.at[pl.ds(tbase + c * WSL, WSL)],
                    sl_a.at[pl.ds(0, WSL)])
    pltpu.sync_copy(sl_a.at[pl.ds(0, WSL)],
                    psrc_out.at[pl.ds(tbase + c * WSL, WSL)])


SUB_ITERS = 8


@functools.partial(
    pl.kernel, mesh=_MESH,
    compiler_params=pltpu.CompilerParams(
        use_tc_tiling_on_sc=False, needs_layout_passes=False),
    out_type=(jax.ShapeDtypeStruct((NP,), jnp.int32),
              jax.ShapeDtypeStruct((LANES,), jnp.int32)),
    scratch_types=[
        pltpu.VMEM((TSL,), jnp.int32),
        pltpu.VMEM((TSL,), jnp.int32),
        pltpu.VMEM((TSL,), jnp.int32),
        pltpu.VMEM((TSL,), jnp.int32),
        pltpu.VMEM((LANES,), jnp.int32),
        pltpu.VMEM((NTILE * LANES,), jnp.int32),
        pltpu.VMEM_SHARED((NP,), jnp.int32),
        pltpu.VMEM_SHARED((NTILE * LANES,), jnp.int32),
    ],
)
def _subtree_kernel(child_hbm, pidx_hbm, s_hbm, out_hbm, cnt_hbm,
                    ch_v, pi_v, s_v, tmp_v, fb_v, fr_v, acc_sh, flag_sh):
    c = lax.axis_index("c")
    s = lax.axis_index("s")
    tbase = s * TSL

    pltpu.sync_copy(child_hbm.at[pl.ds(tbase, TSL)], ch_v)
    pltpu.sync_copy(pidx_hbm.at[pl.ds(tbase, TSL)], pi_v)
    pltpu.sync_copy(s_hbm.at[pl.ds(tbase, TSL)], s_v)

    for it in range(SUB_ITERS):
        _zero_slice(tmp_v, TSL, jnp.int32)
        pltpu.sync_copy(tmp_v, acc_sh.at[pl.ds(tbase, TSL)])
        plsc.subcore_barrier()

        def mul(j, _):
            sl = pl.ds(j * LANES, LANES)
            tmp_v[sl] = ch_v[sl] * s_v[sl]
            return 0
        lax.fori_loop(0, TSL // LANES, mul, 0)
        pltpu.sync_copy(tmp_v, acc_sh.at[pi_v], add=True)
        plsc.subcore_barrier()
        pltpu.sync_copy(acc_sh.at[pl.ds(tbase, TSL)], tmp_v)

        if it < SUB_ITERS - 1:
            def upd(j, _):
                sl = pl.ds(j * LANES, LANES)
                s_v[sl] = 1 + tmp_v[sl]
                return 0
            lax.fori_loop(0, TSL // LANES, upd, 0)
        else:
            def updc(j, cnt):
                sl = pl.ds(j * LANES, LANES)
                sn = 1 + tmp_v[sl]
                df = jnp.sum((sn != s_v[sl]).astype(jnp.int32))
                s_v[sl] = sn
                return cnt + df
            cnt = lax.fori_loop(0, TSL // LANES, updc, jnp.int32(0))
            fb_v[...] = jnp.zeros((LANES,), jnp.int32) + cnt
            pltpu.sync_copy(fb_v, flag_sh.at[pl.ds(s * LANES, LANES)])
            plsc.subcore_barrier()
            pltpu.sync_copy(flag_sh, fr_v)

            def tot(j, acc):
                return acc + jnp.sum(fr_v[pl.ds(j * LANES, LANES)])
            total = lax.fori_loop(0, NTILE, tot, jnp.int32(0))
            fb_v[...] = jnp.zeros((LANES,), jnp.int32) + total

            @pl.when(s == 0)
            def _wcnt():
                pltpu.sync_copy(fb_v.at[pl.ds(0, 8)],
                                cnt_hbm.at[pl.ds(c * 8, 8)])

    pltpu.sync_copy(s_v.at[pl.ds(c * WSL, WSL)],
                    out_hbm.at[pl.ds(tbase + c * WSL, WSL)])


@functools.partial(
    pl.kernel, mesh=_MESH,
    out_type=jax.ShapeDtypeStruct((NP,), jnp.float32),
    scratch_types=[
        pltpu.VMEM((ECH,), jnp.int32),
        pltpu.VMEM((ECH,), jnp.float32),
        pltpu.VMEM((WSL,), jnp.float32),
        pltpu.VMEM_SHARED((NP,), jnp.float32),
    ],
)
def _deg_kernel(src_hbm, out_hbm, src_v, ones_v, sl_v, acc_sh):
    c = lax.axis_index("c")
    s = lax.axis_index("s")
    wid = c * NTILE + s

    _zero_slice(sl_v, WSL, jnp.float32)
    pltpu.sync_copy(sl_v, acc_sh.at[pl.ds(s * (NP // NTILE), WSL)])
    pltpu.sync_copy(sl_v, acc_sh.at[pl.ds(s * (NP // NTILE) + WSL, WSL)])

    pltpu.sync_copy(src_hbm.at[pl.ds(s * ECH, ECH)], src_v)

    def fill(j, _):
        ones_v[pl.ds(j * LANES, LANES)] = jnp.ones((LANES,), jnp.float32)
        return 0
    lax.fori_loop(0, ECH // LANES, fill, 0)

    plsc.subcore_barrier()
    pltpu.sync_copy(ones_v, acc_sh.at[src_v], add=True)
    plsc.subcore_barrier()

    pltpu.sync_copy(acc_sh.at[pl.ds(wid * WSL, WSL)], sl_v)
    pltpu.sync_copy(sl_v, out_hbm.at[pl.ds(wid * WSL, WSL)])


def _struct_feats(edge_index, num_nodes):
    src = edge_index[0].astype(jnp.int32)
    dst = edge_index[1].astype(jnp.int32)
    deg = _deg_kernel(src)[:num_nodes]
    BIG = jnp.iinfo(jnp.int32).max
    n = num_nodes
    pad = jnp.zeros((NP - n,), jnp.int32)

    dist0 = jnp.full((n,), -1, jnp.int32).at[0].set(0)
    parent0 = jnp.full((n,), -1, jnp.int32).at[0].set(0)
    rankx0 = jnp.full((NP,), BIG, jnp.int32).at[0].set(0)
    undisc0 = jnp.concatenate(
        [jnp.ones((n,), jnp.int32).at[0].set(0), pad])

    def bfs_cond(c):
        return c[5] > 0

    def bfs_body(c):
        dist, parent, rankx, undisc, level, _, next_rank = c
        mink_p, psrcp1 = _level_kernel(src, dst, rankx, undisc)
        mink = mink_p[:n]
        psrc = psrcp1[:n] - 1
        new = mink < BIG
        order = jnp.argsort(jnp.where(new, mink, BIG))
        slot = jnp.argsort(order).astype(jnp.int32)
        dist_n = jnp.where(new, level + 1, dist)
        parent_n = jnp.where(new, psrc, parent)
        rankx_n = jnp.concatenate(
            [jnp.where(new, next_rank + slot, BIG), pad + BIG])
        undisc_n = jnp.concatenate(
            [jnp.where(new, 0, undisc[:n]), pad])
        n_new = jnp.sum(new.astype(jnp.int32))
        return (dist_n, parent_n, rankx_n, undisc_n,
                level + 1, n_new, next_rank + n_new)

    dist, parent, _, _, _, _, _ = lax.while_loop(
        bfs_cond, bfs_body,
        (dist0, parent0, rankx0, undisc0,
         jnp.int32(0), jnp.int32(1), jnp.int32(1)))

    max_dist = jnp.max(dist)
    dist = jnp.where(dist < 0, max_dist + 1, dist)

    node_ids = jnp.arange(n, dtype=jnp.int32)
    child = ((parent >= 0) & (parent != node_ids)).astype(jnp.int32)
    pidx = jnp.where(child != 0, parent, 0)
    child_p = jnp.concatenate([child, pad])
    pidx_p = jnp.concatenate([pidx, pad])

    def sub_cond(c):
        return c[1] > 0

    def sub_body(c):
        s, _ = c
        s2, cnt = _subtree_kernel(child_p, pidx_p, s)
        return (s2, jnp.max(cnt))

    s_fix, _ = lax.while_loop(
        sub_cond, sub_body,
        (jnp.ones((NP,), jnp.int32), jnp.int32(1)))
    subtree = s_fix[:n]

    max_sub = jnp.max(subtree)
    dist_t = dist.astype(jnp.float32)
    sub_t = subtree.astype(jnp.float32)
    dist_norm = jnp.where(
        max_dist > 0,
        dist_t / jnp.where(max_dist > 0, max_dist, 1).astype(jnp.float32),
        dist_t)
    sub_norm = jnp.where(
        max_sub > 0,
        sub_t / jnp.where(max_sub > 0, max_sub, 1).astype(jnp.float32),
        sub_t)
    max_deg = jnp.max(deg)
    deg_norm = jnp.where(
        max_deg > 0,
        deg / jnp.where(max_deg > 0, max_deg, 1.0),
        jnp.zeros_like(deg))
    return dist_norm, sub_norm, deg_norm


def _readout_body(win_ref, emb_ref, w1_ref, b1_ref, w2_ref, b2_ref,
                  out_ref, m_ref, s_ref, acc_ref):
    i = pl.program_id(0)

    @pl.when(i == 0)
    def _init():
        m_ref[0, 0] = -jnp.inf
        s_ref[0, 0] = 0.0
        acc_ref[...] = jnp.zeros_like(acc_ref)

    x = win_ref[...]
    h = jnp.maximum(
        jnp.dot(x, w1_ref[...], preferred_element_type=jnp.float32)
        + b1_ref[...], 0.0)
    z = (jnp.dot(h, w2_ref[...], preferred_element_type=jnp.float32)
         + b2_ref[0, 0])
    z = z[:, 0]
    m_old = m_ref[0, 0]
    m_new = jnp.maximum(m_old, jnp.max(z))
    corr = jnp.exp(m_old - m_new)
    e = jnp.exp(z - m_new)
    s_ref[0, 0] = s_ref[0, 0] * corr + jnp.sum(e)
    acc_ref[...] = acc_ref[...] * corr + jnp.dot(
        e[None, :], emb_ref[...], preferred_element_type=jnp.float32)
    m_ref[0, 0] = m_new

    @pl.when(i == N_BLOCKS - 1)
    def _fin():
        out_ref[...] = acc_ref[...] / s_ref[0, 0]


def _weighted_readout(weight_in, node_embed, W1T, b1, W2T, b2):
    return pl.pallas_call(
        _readout_body,
        grid=(N_BLOCKS,),
        in_specs=[
            pl.BlockSpec((ROW_BLOCK, IN_DIM), lambda i: (i, 0)),
            pl.BlockSpec((ROW_BLOCK, D_EMBED), lambda i: (i, 0)),
            pl.BlockSpec((IN_DIM, HIDDEN), lambda i: (0, 0)),
            pl.BlockSpec((1, HIDDEN), lambda i: (0, 0)),
            pl.BlockSpec((HIDDEN, 1), lambda i: (0, 0)),
            pl.BlockSpec((1, 1), lambda i: (0, 0), memory_space=pltpu.SMEM),
        ],
        out_specs=pl.BlockSpec((1, D_EMBED), lambda i: (0, 0)),
        out_shape=jax.ShapeDtypeStruct((1, D_EMBED), jnp.float32),
        scratch_shapes=[
            pltpu.SMEM((1, 1), jnp.float32),
            pltpu.SMEM((1, 1), jnp.float32),
            pltpu.VMEM((1, D_EMBED), jnp.float32),
        ],
    )(weight_in, node_embed, W1T, b1, W2T, b2)


def kernel(node_embed, data, attr_x, edge_index, W1, b1, W2, b2):
    num_nodes = node_embed.shape[0]
    dist_norm, sub_norm, deg_norm = _struct_feats(edge_index, num_nodes)
    struct = jnp.stack([1.0 - dist_norm, sub_norm, deg_norm], axis=1)
    attr = attr_x[:, -ATTR_DIM:]
    weight_in = jnp.concatenate([attr, struct], axis=1)
    out = _weighted_readout(
        weight_in, node_embed,
        W1.T, b1.reshape(1, HIDDEN), W2.T, b2.reshape(1, 1))
    return out


# speedup vs baseline: 1.9414x; 1.0542x over previous
import functools

import jax
import jax.numpy as jnp
from jax import lax
from jax.experimental import pallas as pl
from jax.experimental.pallas import tpu as pltpu
from jax.experimental.pallas import tpu_sc as plsc

N_NODES = 10000
N_EDGES = 160000
D_EMBED = 256
ATTR_DIM = 16
HIDDEN = 64
IN_DIM = ATTR_DIM + 3

ROW_BLOCK = 1000
N_BLOCKS = N_NODES // ROW_BLOCK

NTILE = 16
ECH = N_EDGES // NTILE
NP = 10240
WSL = NP // 32
LANES = 16

_MESH = plsc.VectorSubcoreMesh(core_axis_name="c", subcore_axis_name="s")


BIGI = jnp.iinfo(jnp.int32).max
TSL = NP // NTILE


def _zero_slice(buf, n, dtype):
    def body(j, _):
        buf[pl.ds(j * LANES, LANES)] = jnp.zeros((LANES,), dtype)
        return 0
    lax.fori_loop(0, n // LANES, body, 0)


def _fill_slice(buf, n, value):
    def body(j, _):
        buf[pl.ds(j * LANES, LANES)] = jnp.full((LANES,), value, jnp.int32)
        return 0
    lax.fori_loop(0, n // LANES, body, 0)


_GDN = lax.GatherDimensionNumbers(
    offset_dims=(), collapsed_slice_dims=(0,), start_index_map=(0,))


def _lane_gather(v, idx):
    return lax.gather(v, idx[:, None], _GDN, (1,),
                      mode=lax.GatherScatterMode.PROMISE_IN_BOUNDS)


@functools.partial(
    pl.kernel, mesh=_MESH,
    compiler_params=pltpu.CompilerParams(
        use_tc_tiling_on_sc=False, needs_layout_passes=False),
    out_type=(jax.ShapeDtypeStruct((NP,), jnp.int32),
              jax.ShapeDtypeStruct((NP,), jnp.int32)),
    scratch_types=[
        pltpu.VMEM((ECH,), jnp.int32),
        pltpu.VMEM((ECH,), jnp.int32),
        pltpu.VMEM((ECH,), jnp.int32),
        pltpu.VMEM((ECH,), jnp.int32),
        pltpu.VMEM((NP,), jnp.int32),
        pltpu.VMEM((NP,), jnp.int32),
        pltpu.VMEM((NP,), jnp.int32),
        pltpu.VMEM((TSL,), jnp.int32),
        pltpu.VMEM((NTILE, TSL), jnp.int32),
        pltpu.VMEM_SHARED((NTILE, NP), jnp.int32),
        pltpu.VMEM_SHARED((NP,), jnp.int32),
        pltpu.VMEM_SHARED((NP,), jnp.int32),
        pltpu.SemaphoreType.DMA,
    ],
)
def _level_kernel(src_hbm, dst_hbm, rankx_hbm, undisc_hbm,
                  mink_out, psrc_out,
                  src_v, dst_v, key_v, val_v, rankx_v, undisc_v, mink_v,
                  sl_a, red_v, pub_sh, minkg_sh, pacc_sh, sem):
    c = lax.axis_index("c")
    s = lax.axis_index("s")
    tbase = s * TSL
    ii = lax.iota(jnp.int32, LANES)

    cp1 = pltpu.async_copy(src_hbm.at[pl.ds(s * ECH, ECH)], src_v, sem)
    cp2 = pltpu.async_copy(dst_hbm.at[pl.ds(s * ECH, ECH)], dst_v, sem)
    cp3 = pltpu.async_copy(rankx_hbm, rankx_v, sem)
    cp4 = pltpu.async_copy(undisc_hbm, undisc_v, sem)

    _fill_slice(mink_v, NP, BIGI)
    _zero_slice(sl_a, TSL, jnp.int32)
    pltpu.sync_copy(sl_a, pacc_sh.at[pl.ds(tbase, TSL)])
    cp1.wait()
    cp2.wait()
    cp3.wait()
    cp4.wait()

    @plsc.parallel_loop(0, ECH // LANES, unroll=4)
    def pA(i):
        sl = pl.ds(i * LANES, LANES)
        rk = plsc.load_gather(rankx_v, [src_v[sl]])
        u = plsc.load_gather(undisc_v, [dst_v[sl]])
        cand = (rk != BIGI) & (u != 0)
        pos = (s * ECH + i * LANES) + ii
        key_v[sl] = jnp.where(cand, rk * N_EDGES + pos, BIGI)

    def pB(i, _):
        sl = pl.ds(i * LANES, LANES)
        kk = key_v[sl]

        @pl.when(jnp.min(kk) != BIGI)
        def _active():
            dd, ks = plsc.sort_key_val(dst_v[sl], kk)
            rm = ks
            for sh in (1, 2, 4, 8):
                idxu = jnp.maximum(ii - sh, 0)
                d_sh = _lane_gather(dd, idxu)
                m_sh = _lane_gather(rm, idxu)
                same = (ii >= sh) & (d_sh == dd)
                rm = jnp.minimum(rm, jnp.where(same, m_sh, BIGI))
            d_nx = _lane_gather(dd, jnp.minimum(ii + 1, LANES - 1))
            is_last = (ii == LANES - 1) | (d_nx != dd)
            cur = plsc.load_gather(mink_v, [dd])
            plsc.store_scatter(mink_v, [dd], jnp.minimum(cur, rm),
                               mask=is_last)
        return 0

    lax.fori_loop(0, ECH // LANES, pB, 0)

    pltpu.sync_copy(mink_v, pub_sh.at[s])
    plsc.subcore_barrier()
    pltpu.sync_copy(pub_sh.at[:, pl.ds(tbase, TSL)], red_v)

    def red(j, _):
        sl = pl.ds(j * LANES, LANES)
        m = red_v[0, sl]
        for r in range(1, NTILE):
            m = jnp.minimum(m, red_v[r, sl])
        sl_a[sl] = m
        return 0
    lax.fori_loop(0, TSL // LANES, red, 0)

    pltpu.sync_copy(sl_a, minkg_sh.at[pl.ds(tbase, TSL)])
    cpo = pltpu.async_copy(sl_a.at[pl.ds(c * WSL, WSL)],
                           mink_out.at[pl.ds(tbase + c * WSL, WSL)], sem)
    plsc.subcore_barrier()
    pltpu.sync_copy(minkg_sh, mink_v)

    @plsc.parallel_loop(0, ECH // LANES, unroll=2)
    def pC(i):
        sl = pl.ds(i * LANES, LANES)
        kk = key_v[sl]
        mg = plsc.load_gather(mink_v, [dst_v[sl]])
        win = (kk != BIGI) & (kk == mg)
        val_v[sl] = jnp.where(win, src_v[sl] + 1, 0)

    cpo.wait()
    pltpu.sync_copy(val_v, pacc_sh.at[dst_v], add=True)
    plsc.subcore_barrier()
    pltpu.sync_copy(pacc_sh.at[pl.ds(tbase + c * WSL, WSL)],
                    sl_a.at[pl.ds(0, WSL)])
    pltpu.sync_copy(sl_a.at[pl.ds(0, WSL)],
                    psrc_out.at[pl.ds(tbase + c * WSL, WSL)])


SUB_ITERS = 8


@functools.partial(
    pl.kernel, mesh=_MESH,
    compiler_params=pltpu.CompilerParams(
        use_tc_tiling_on_sc=False, needs_layout_passes=False),
    out_type=(jax.ShapeDtypeStruct((NP,), jnp.int32),
              jax.ShapeDtypeStruct((LANES,), jnp.int32)),
    scratch_types=[
        pltpu.VMEM((TSL,), jnp.int32),
        pltpu.VMEM((TSL,), jnp.int32),
        pltpu.VMEM((TSL,), jnp.int32),
        pltpu.VMEM((TSL,), jnp.int32),
        pltpu.VMEM((LANES,), jnp.int32),
        pltpu.VMEM((NTILE * LANES,), jnp.int32),
        pltpu.VMEM_SHARED((NP,), jnp.int32),
        pltpu.VMEM_SHARED((NTILE * LANES,), jnp.int32),
    ],
)
def _subtree_kernel(child_hbm, pidx_hbm, s_hbm, out_hbm, cnt_hbm,
                    ch_v, pi_v, s_v, tmp_v, fb_v, fr_v, acc_sh, flag_sh):
    c = lax.axis_index("c")
    s = lax.axis_index("s")
    tbase = s * TSL

    pltpu.sync_copy(child_hbm.at[pl.ds(tbase, TSL)], ch_v)
    pltpu.sync_copy(pidx_hbm.at[pl.ds(tbase, TSL)], pi_v)
    pltpu.sync_copy(s_hbm.at[pl.ds(tbase, TSL)], s_v)

    for it in range(SUB_ITERS):
        _zero_slice(tmp_v, TSL, jnp.int32)
        pltpu.sync_copy(tmp_v, acc_sh.at[pl.ds(tbase, TSL)])
        plsc.subcore_barrier()

        def mul(j, _):
            sl = pl.ds(j * LANES, LANES)
            tmp_v[sl] = ch_v[sl] * s_v[sl]
            return 0
        lax.fori_loop(0, TSL // LANES, mul, 0)
        pltpu.sync_copy(tmp_v, acc_sh.at[pi_v], add=True)
        plsc.subcore_barrier()
        pltpu.sync_copy(acc_sh.at[pl.ds(tbase, TSL)], tmp_v)

        if it < SUB_ITERS - 1:
            def upd(j, _):
                sl = pl.ds(j * LANES, LANES)
                s_v[sl] = 1 + tmp_v[sl]
                return 0
            lax.fori_loop(0, TSL // LANES, upd, 0)
        else:
            def updc(j, cnt):
                sl = pl.ds(j * LANES, LANES)
                sn = 1 + tmp_v[sl]
                df = jnp.sum((sn != s_v[sl]).astype(jnp.int32))
                s_v[sl] = sn
                return cnt + df
            cnt = lax.fori_loop(0, TSL // LANES, updc, jnp.int32(0))
            fb_v[...] = jnp.zeros((LANES,), jnp.int32) + cnt
            pltpu.sync_copy(fb_v, flag_sh.at[pl.ds(s * LANES, LANES)])
            plsc.subcore_barrier()
            pltpu.sync_copy(flag_sh, fr_v)

            def tot(j, acc):
                return acc + jnp.sum(fr_v[pl.ds(j * LANES, LANES)])
            total = lax.fori_loop(0, NTILE, tot, jnp.int32(0))
            fb_v[...] = jnp.zeros((LANES,), jnp.int32) + total

            @pl.when(s == 0)
            def _wcnt():
                pltpu.sync_copy(fb_v.at[pl.ds(0, 8)],
                                cnt_hbm.at[pl.ds(c * 8, 8)])

    pltpu.sync_copy(s_v.at[pl.ds(c * WSL, WSL)],
                    out_hbm.at[pl.ds(tbase + c * WSL, WSL)])


@functools.partial(
    pl.kernel, mesh=_MESH,
    out_type=jax.ShapeDtypeStruct((NP,), jnp.float32),
    scratch_types=[
        pltpu.VMEM((ECH,), jnp.int32),
        pltpu.VMEM((ECH,), jnp.float32),
        pltpu.VMEM((WSL,), jnp.float32),
        pltpu.VMEM_SHARED((NP,), jnp.float32),
    ],
)
def _deg_kernel(src_hbm, out_hbm, src_v, ones_v, sl_v, acc_sh):
    c = lax.axis_index("c")
    s = lax.axis_index("s")
    wid = c * NTILE + s

    _zero_slice(sl_v, WSL, jnp.float32)
    pltpu.sync_copy(sl_v, acc_sh.at[pl.ds(s * (NP // NTILE), WSL)])
    pltpu.sync_copy(sl_v, acc_sh.at[pl.ds(s * (NP // NTILE) + WSL, WSL)])

    pltpu.sync_copy(src_hbm.at[pl.ds(s * ECH, ECH)], src_v)

    def fill(j, _):
        ones_v[pl.ds(j * LANES, LANES)] = jnp.ones((LANES,), jnp.float32)
        return 0
    lax.fori_loop(0, ECH // LANES, fill, 0)

    plsc.subcore_barrier()
    pltpu.sync_copy(ones_v, acc_sh.at[src_v], add=True)
    plsc.subcore_barrier()

    pltpu.sync_copy(acc_sh.at[pl.ds(wid * WSL, WSL)], sl_v)
    pltpu.sync_copy(sl_v, out_hbm.at[pl.ds(wid * WSL, WSL)])


def _struct_feats(edge_index, num_nodes):
    src = edge_index[0].astype(jnp.int32)
    dst = edge_index[1].astype(jnp.int32)
    deg = _deg_kernel(src)[:num_nodes]
    BIG = jnp.iinfo(jnp.int32).max
    n = num_nodes
    pad = jnp.zeros((NP - n,), jnp.int32)

    dist0 = jnp.full((n,), -1, jnp.int32).at[0].set(0)
    parent0 = jnp.full((n,), -1, jnp.int32).at[0].set(0)
    rankx0 = jnp.full((NP,), BIG, jnp.int32).at[0].set(0)
    undisc0 = jnp.concatenate(
        [jnp.ones((n,), jnp.int32).at[0].set(0), pad])

    def bfs_cond(c):
        return c[5] > 0

    def bfs_body(c):
        dist, parent, rankx, undisc, level, _, next_rank = c
        mink_p, psrcp1 = _level_kernel(src, dst, rankx, undisc)
        mink = mink_p[:n]
        psrc = psrcp1[:n] - 1
        new = mink < BIG
        order = jnp.argsort(jnp.where(new, mink, BIG))
        slot = jnp.argsort(order).astype(jnp.int32)
        dist_n = jnp.where(new, level + 1, dist)
        parent_n = jnp.where(new, psrc, parent)
        rankx_n = jnp.concatenate(
            [jnp.where(new, next_rank + slot, BIG), pad + BIG])
        undisc_n = jnp.concatenate(
            [jnp.where(new, 0, undisc[:n]), pad])
        n_new = jnp.sum(new.astype(jnp.int32))
        return (dist_n, parent_n, rankx_n, undisc_n,
                level + 1, n_new, next_rank + n_new)

    dist, parent, _, _, _, _, _ = lax.while_loop(
        bfs_cond, bfs_body,
        (dist0, parent0, rankx0, undisc0,
         jnp.int32(0), jnp.int32(1), jnp.int32(1)))

    max_dist = jnp.max(dist)
    dist = jnp.where(dist < 0, max_dist + 1, dist)

    node_ids = jnp.arange(n, dtype=jnp.int32)
    child = ((parent >= 0) & (parent != node_ids)).astype(jnp.int32)
    pidx = jnp.where(child != 0, parent, 0)
    child_p = jnp.concatenate([child, pad])
    pidx_p = jnp.concatenate([pidx, pad])

    def sub_cond(c):
        return c[1] > 0

    def sub_body(c):
        s, _ = c
        s2, cnt = _subtree_kernel(child_p, pidx_p, s)
        return (s2, jnp.max(cnt))

    s_fix, _ = lax.while_loop(
        sub_cond, sub_body,
        (jnp.ones((NP,), jnp.int32), jnp.int32(1)))
    subtree = s_fix[:n]

    max_sub = jnp.max(subtree)
    dist_t = dist.astype(jnp.float32)
    sub_t = subtree.astype(jnp.float32)
    dist_norm = jnp.where(
        max_dist > 0,
        dist_t / jnp.where(max_dist > 0, max_dist, 1).astype(jnp.float32),
        dist_t)
    sub_norm = jnp.where(
        max_sub > 0,
        sub_t / jnp.where(max_sub > 0, max_sub, 1).astype(jnp.float32),
        sub_t)
    max_deg = jnp.max(deg)
    deg_norm = jnp.where(
        max_deg > 0,
        deg / jnp.where(max_deg > 0, max_deg, 1.0),
        jnp.zeros_like(deg))
    return dist_norm, sub_norm, deg_norm


def _readout_body(win_ref, emb_ref, w1_ref, b1_ref, w2_ref, b2_ref,
                  out_ref, m_ref, s_ref, acc_ref):
    i = pl.program_id(0)

    @pl.when(i == 0)
    def _init():
        m_ref[0, 0] = -jnp.inf
        s_ref[0, 0] = 0.0
        acc_ref[...] = jnp.zeros_like(acc_ref)

    x = win_ref[...]
    h = jnp.maximum(
        jnp.dot(x, w1_ref[...], preferred_element_type=jnp.float32)
        + b1_ref[...], 0.0)
    z = (jnp.dot(h, w2_ref[...], preferred_element_type=jnp.float32)
         + b2_ref[0, 0])
    z = z[:, 0]
    m_old = m_ref[0, 0]
    m_new = jnp.maximum(m_old, jnp.max(z))
    corr = jnp.exp(m_old - m_new)
    e = jnp.exp(z - m_new)
    s_ref[0, 0] = s_ref[0, 0] * corr + jnp.sum(e)
    acc_ref[...] = acc_ref[...] * corr + jnp.dot(
        e[None, :], emb_ref[...], preferred_element_type=jnp.float32)
    m_ref[0, 0] = m_new

    @pl.when(i == N_BLOCKS - 1)
    def _fin():
        out_ref[...] = acc_ref[...] / s_ref[0, 0]


def _weighted_readout(weight_in, node_embed, W1T, b1, W2T, b2):
    return pl.pallas_call(
        _readout_body,
        grid=(N_BLOCKS,),
        in_specs=[
            pl.BlockSpec((ROW_BLOCK, IN_DIM), lambda i: (i, 0)),
            pl.BlockSpec((ROW_BLOCK, D_EMBED), lambda i: (i, 0)),
            pl.BlockSpec((IN_DIM, HIDDEN), lambda i: (0, 0)),
            pl.BlockSpec((1, HIDDEN), lambda i: (0, 0)),
            pl.BlockSpec((HIDDEN, 1), lambda i: (0, 0)),
            pl.BlockSpec((1, 1), lambda i: (0, 0), memory_space=pltpu.SMEM),
        ],
        out_specs=pl.BlockSpec((1, D_EMBED), lambda i: (0, 0)),
        out_shape=jax.ShapeDtypeStruct((1, D_EMBED), jnp.float32),
        scratch_shapes=[
            pltpu.SMEM((1, 1), jnp.float32),
            pltpu.SMEM((1, 1), jnp.float32),
            pltpu.VMEM((1, D_EMBED), jnp.float32),
        ],
    )(weight_in, node_embed, W1T, b1, W2T, b2)


def kernel(node_embed, data, attr_x, edge_index, W1, b1, W2, b2):
    num_nodes = node_embed.shape[0]
    dist_norm, sub_norm, deg_norm = _struct_feats(edge_index, num_nodes)
    struct = jnp.stack([1.0 - dist_norm, sub_norm, deg_norm], axis=1)
    attr = attr_x[:, -ATTR_DIM:]
    weight_in = jnp.concatenate([attr, struct], axis=1)
    out = _weighted_readout(
        weight_in, node_embed,
        W1.T, b1.reshape(1, HIDDEN), W2.T, b2.reshape(1, 1))
    return out
